# baseline (speedup 1.0000x reference)
import jax
import jax.numpy as jnp
from jax import lax
from jax.experimental import pallas as pl
from jax.experimental.pallas import tpu as pltpu
from jax.experimental.pallas import tpu_sc as plsc

B, L, D = 4096, 200, 64
N = B * L
NC, NS = 2, 16
NW = NC * NS
ROWS_PER_W = N // NW
GATHER = 128
CHUNK = 512
G_PER_CHUNK = CHUNK // GATHER
N_ITERS = ROWS_PER_W // CHUNK


def _emb_body(idx_hbm, table_hbm, out_hbm, idx_v, rows_v, row_sem):
    wid = lax.axis_index("s") * NC + lax.axis_index("c")
    idx_base = wid * (ROWS_PER_W // GATHER)
    out_base = wid * ROWS_PER_W

    def body(t, carry):
        pltpu.sync_copy(idx_hbm.at[pl.ds(idx_base + t * G_PER_CHUNK,
                                         G_PER_CHUNK)], idx_v)
        cps = [pltpu.async_copy(table_hbm.at[idx_v.at[j]],
                                rows_v.at[pl.ds(j * GATHER, GATHER)],
                                row_sem)
               for j in range(G_PER_CHUNK)]
        for c in cps:
            c.wait()
        pltpu.sync_copy(rows_v,
                        out_hbm.at[pl.ds(out_base + t * CHUNK, CHUNK)])
        return carry

    lax.fori_loop(0, N_ITERS, body, 0)


def kernel(x, W):
    idx = x.reshape(N // GATHER, GATHER).astype(jnp.int32)
    mesh = plsc.VectorSubcoreMesh(core_axis_name="c", subcore_axis_name="s")
    run = pl.kernel(
        _emb_body,
        out_type=jax.ShapeDtypeStruct((N, D), jnp.float32),
        mesh=mesh,
        scratch_types=[
            pltpu.VMEM((G_PER_CHUNK, GATHER), jnp.int32),
            pltpu.VMEM((CHUNK, D), jnp.float32),
            pltpu.SemaphoreType.DMA,
        ],
        compiler_params=pltpu.CompilerParams(use_tc_tiling_on_sc=False),
    )
    out = run(idx, W)
    return out.reshape(B, L, D)


# speedup vs baseline: 1.0386x; 1.0386x over previous
import jax
import jax.numpy as jnp
from jax import lax
from jax.experimental import pallas as pl
from jax.experimental.pallas import tpu as pltpu
from jax.experimental.pallas import tpu_sc as plsc

B, L, D = 4096, 200, 64
N = B * L
NC, NS = 2, 16
NW = NC * NS
ROWS_PER_W = N // NW
GATHER = 128
CHUNK = 512
G_PER_CHUNK = CHUNK // GATHER
N_ITERS = ROWS_PER_W // CHUNK
IDX_ROWS = ROWS_PER_W // GATHER


def _emb_body(idx_hbm, table_hbm, out_hbm, idx_v, rows_v,
              g_sem0, g_sem1, s_sem0, s_sem1):
    wid = lax.axis_index("s") * NC + lax.axis_index("c")
    out_base = wid * ROWS_PER_W
    g_sems = (g_sem0, g_sem1)
    s_sems = (s_sem0, s_sem1)

    def issue_gathers(t, buf):
        for j in range(G_PER_CHUNK):
            pltpu.async_copy(table_hbm.at[idx_v.at[t * G_PER_CHUNK + j]],
                             rows_v.at[buf, pl.ds(j * GATHER, GATHER)],
                             g_sems[buf])

    def wait_gathers(buf):
        for j in range(G_PER_CHUNK):
            pltpu.make_async_copy(table_hbm.at[idx_v.at[j]],
                                  rows_v.at[buf, pl.ds(j * GATHER, GATHER)],
                                  g_sems[buf]).wait()

    def issue_store(t, buf):
        pltpu.async_copy(rows_v.at[buf],
                         out_hbm.at[pl.ds(out_base + t * CHUNK, CHUNK)],
                         s_sems[buf])

    def wait_store(buf):
        pltpu.make_async_copy(rows_v.at[buf],
                              out_hbm.at[pl.ds(out_base, CHUNK)],
                              s_sems[buf]).wait()

    pltpu.sync_copy(idx_hbm.at[pl.ds(wid * IDX_ROWS, IDX_ROWS)], idx_v)

    issue_gathers(0, 0)
    issue_gathers(1, 1)

    def body(tt, carry):
        t0 = tt * 2
        t1 = t0 + 1
        wait_gathers(0)
        issue_store(t0 - 2, 0)
        wait_gathers(1)
        issue_store(t1 - 2, 1)
        wait_store(0)
        issue_gathers(t0, 0)
        wait_store(1)
        issue_gathers(t1, 1)
        return carry

    lax.fori_loop(1, N_ITERS // 2, body, 0)

    wait_gathers(0)
    issue_store(N_ITERS - 2, 0)
    wait_gathers(1)
    issue_store(N_ITERS - 1, 1)
    wait_store(0)
    wait_store(1)


def kernel(x, W):
    idx = x.reshape(N // GATHER, GATHER).astype(jnp.int32)
    mesh = plsc.VectorSubcoreMesh(core_axis_name="c", subcore_axis_name="s")
    run = pl.kernel(
        _emb_body,
        out_type=jax.ShapeDtypeStruct((N, D), jnp.float32),
        mesh=mesh,
        scratch_types=[
            pltpu.VMEM((IDX_ROWS, GATHER), jnp.int32),
            pltpu.VMEM((2, CHUNK, D), jnp.float32),
            pltpu.SemaphoreType.DMA,
            pltpu.SemaphoreType.DMA,
            pltpu.SemaphoreType.DMA,
            pltpu.SemaphoreType.DMA,
        ],
        compiler_params=pltpu.CompilerParams(use_tc_tiling_on_sc=False),
    )
    out = run(idx, W)
    return out.reshape(B, L, D)
